# baseline (speedup 1.0000x reference)
import functools

import jax
import jax.numpy as jnp
from jax import lax
from jax.experimental import pallas as pl
from jax.experimental.pallas import tpu as pltpu
from jax.experimental.pallas import tpu_sc as plsc

B, T, D = 16, 576, 256
KSIZE = 1024
NROW = B * T


BR = 3072
CH = 256


def _tc_body(f_ref, k_ref, idx_ref, conf_ref):
    f = f_ref[...]
    big = jnp.float32(jnp.inf)

    m1 = jnp.full((BR,), big)
    i1 = jnp.full((BR,), KSIZE, jnp.int32)
    m2 = jnp.full((BR,), big)
    for j in range(KSIZE // CH):
        kc = k_ref[j * CH:(j + 1) * CH, :]
        s = lax.dot_general(
            kc, f, (((1,), (1,)), ((), ())),
            preferred_element_type=jnp.float32,
            precision=lax.Precision.DEFAULT,
        )
        knc = jnp.sum(kc * kc, axis=1, keepdims=True)
        s = knc - 2.0 * s
        cm1 = jnp.min(s, axis=0)
        col = lax.broadcasted_iota(jnp.int32, (CH, BR), 0) + j * CH
        ci1 = jnp.min(jnp.where(s == cm1[None, :], col, KSIZE), axis=0)
        s2 = jnp.where(col == ci1[None, :], big, s)
        cm2 = jnp.min(s2, axis=0)
        i1 = jnp.where(cm1 < m1, ci1, i1)
        m2 = jnp.minimum(jnp.maximum(m1, cm1), jnp.minimum(m2, cm2))
        m1 = jnp.minimum(m1, cm1)

    fn = jnp.sum(f * f, axis=1)
    d1 = jnp.sqrt(jnp.maximum(m1 + fn, 1e-12))
    d2 = jnp.sqrt(jnp.maximum(m2 + fn, 1e-12))
    idx_ref[...] = i1
    conf = 1.0 - d1 / (d2 + 1e-8)
    conf_ref[...] = jnp.broadcast_to(conf[:, None], (BR, 16))


def _tc_top2(f, K, interpret=False):
    grid = NROW // BR
    return pl.pallas_call(
        _tc_body,
        grid=(grid,),
        in_specs=[
            pl.BlockSpec((BR, D), lambda i: (i, 0)),
            pl.BlockSpec((KSIZE, D), lambda i: (0, 0)),
        ],
        out_specs=[
            pl.BlockSpec((BR,), lambda i: (i,)),
            pl.BlockSpec((BR, 16), lambda i: (i, 0)),
        ],
        out_shape=[
            jax.ShapeDtypeStruct((NROW,), jnp.int32),
            jax.ShapeDtypeStruct((NROW, 16), jnp.float32),
        ],
        interpret=interpret,
    )(f, K)


NC, NS, L = 2, 16, 16
NW = NC * NS
RPW = NROW // NW
DCH = D // L


NCHUNK = 4
CR = RPW // NCHUNK


def _sc_body(k_hbm, idx_hbm, conf_hbm, out_hbm, idx_v, conf_v, rows_v,
             g0, g1, osem):
    wid = lax.axis_index("s") * NC + lax.axis_index("c")
    base = wid * RPW
    pltpu.sync_copy(idx_hbm.at[pl.ds(base, RPW)], idx_v)
    pltpu.sync_copy(conf_hbm.at[pl.ds(base, RPW)], conf_v)

    gsem = (g0, g1)

    def gather(c):
        return pltpu.async_copy(
            k_hbm.at[idx_v.at[pl.ds(c * CR, CR)]], rows_v.at[c],
            gsem[c % 2])

    pending = gather(0)
    out_handles = []
    for c in range(NCHUNK):
        nxt = gather(c + 1) if c + 1 < NCHUNK else None
        pending.wait()
        pending = nxt

        def scale_row(r, carry, c=c):
            cv = conf_v[c * CR + r, :]
            for d in range(DCH):
                sl = pl.ds(d * L, L)
                rows_v[c, r, sl] = rows_v[c, r, sl] * cv
            return carry

        lax.fori_loop(0, CR, scale_row, 0)
        out_handles.append(pltpu.async_copy(
            rows_v.at[c], out_hbm.at[pl.ds(base + c * CR, CR)], osem))
    for h in out_handles:
        h.wait()


@functools.cache
def _sc_gather_scale():
    return pl.kernel(
        _sc_body,
        out_type=jax.ShapeDtypeStruct((NROW, D), jnp.float32),
        mesh=plsc.VectorSubcoreMesh(core_axis_name="c", subcore_axis_name="s"),
        scratch_types=[
            pltpu.VMEM((RPW,), jnp.int32),
            pltpu.VMEM((RPW, L), jnp.float32),
            pltpu.VMEM((NCHUNK, CR, D), jnp.float32),
            pltpu.SemaphoreType.DMA,
            pltpu.SemaphoreType.DMA,
            pltpu.SemaphoreType.DMA,
        ],
    )


def kernel(f_ipm, K):
    orig_shape = f_ipm.shape
    f = f_ipm.reshape(-1, orig_shape[-1])
    idx, conf = _tc_top2(f, K)
    out = _sc_gather_scale()(K, idx, conf)
    return out.reshape(orig_shape)


# speedup vs baseline: 1.0816x; 1.0816x over previous
import functools

import jax
import jax.numpy as jnp
from jax import lax
from jax.experimental import pallas as pl
from jax.experimental.pallas import tpu as pltpu
from jax.experimental.pallas import tpu_sc as plsc

B, T, D = 16, 576, 256
KSIZE = 1024
NROW = B * T


BR = 3072
CH = 256


def _tc_body(f_ref, k_ref, idx_ref, conf_ref):
    f = f_ref[...]
    big = jnp.float32(jnp.inf)

    m1 = jnp.full((BR,), big)
    i1 = jnp.full((BR,), KSIZE, jnp.int32)
    m2 = jnp.full((BR,), big)
    for j in range(KSIZE // CH):
        kc = k_ref[j * CH:(j + 1) * CH, :]
        s = lax.dot_general(
            kc, f, (((1,), (1,)), ((), ())),
            preferred_element_type=jnp.float32,
            precision=lax.Precision.DEFAULT,
        )
        knc = jnp.sum(kc * kc, axis=1, keepdims=True)
        s = knc - 2.0 * s
        cm1 = jnp.min(s, axis=0)
        col = lax.broadcasted_iota(jnp.int32, (CH, BR), 0) + j * CH
        ci1 = jnp.min(jnp.where(s == cm1[None, :], col, KSIZE), axis=0)
        s2 = jnp.where(col == ci1[None, :], big, s)
        cm2 = jnp.min(s2, axis=0)
        i1 = jnp.where(cm1 < m1, ci1, i1)
        m2 = jnp.minimum(jnp.maximum(m1, cm1), jnp.minimum(m2, cm2))
        m1 = jnp.minimum(m1, cm1)

    fn = jnp.sum(f * f, axis=1)
    d1 = jnp.sqrt(jnp.maximum(m1 + fn, 1e-12))
    d2 = jnp.sqrt(jnp.maximum(m2 + fn, 1e-12))
    idx_ref[...] = i1
    conf = 1.0 - d1 / (d2 + 1e-8)
    conf_ref[...] = jnp.broadcast_to(conf[:, None], (BR, 16))


def _tc_top2(f, K, interpret=False):
    grid = NROW // BR
    return pl.pallas_call(
        _tc_body,
        grid=(grid,),
        in_specs=[
            pl.BlockSpec((BR, D), lambda i: (i, 0)),
            pl.BlockSpec((KSIZE, D), lambda i: (0, 0)),
        ],
        out_specs=[
            pl.BlockSpec((BR,), lambda i: (i,)),
            pl.BlockSpec((BR, 16), lambda i: (i, 0)),
        ],
        out_shape=[
            jax.ShapeDtypeStruct((NROW,), jnp.int32),
            jax.ShapeDtypeStruct((NROW, 16), jnp.float32),
        ],
        interpret=interpret,
    )(f, K)


NC, NS, L = 2, 16, 16
NW = NC * NS
RPW = NROW // NW
DCH = D // L


def _sc_body(k_hbm, idx_hbm, conf_hbm, out_hbm, idx_v, conf_v, rows_v, sem):
    wid = lax.axis_index("s") * NC + lax.axis_index("c")
    base = wid * RPW
    pltpu.sync_copy(idx_hbm.at[pl.ds(base, RPW)], idx_v)
    pltpu.sync_copy(conf_hbm.at[pl.ds(base, RPW)], conf_v)
    pltpu.async_copy(k_hbm.at[idx_v], rows_v, sem).wait()

    @plsc.parallel_loop(0, RPW, 1, unroll=2)
    def _(r):
        cv = conf_v[r, :]
        for d in range(DCH):
            sl = pl.ds(d * L, L)
            rows_v[r, sl] = rows_v[r, sl] * cv

    pltpu.sync_copy(rows_v, out_hbm.at[pl.ds(base, RPW)])


@functools.cache
def _sc_gather_scale():
    return pl.kernel(
        _sc_body,
        out_type=jax.ShapeDtypeStruct((NROW, D), jnp.float32),
        mesh=plsc.VectorSubcoreMesh(core_axis_name="c", subcore_axis_name="s"),
        scratch_types=[
            pltpu.VMEM((RPW,), jnp.int32),
            pltpu.VMEM((RPW, L), jnp.float32),
            pltpu.VMEM((RPW, D), jnp.float32),
            pltpu.SemaphoreType.DMA,
        ],
    )


def kernel(f_ipm, K):
    orig_shape = f_ipm.shape
    f = f_ipm.reshape(-1, orig_shape[-1])
    idx, conf = _tc_top2(f, K)
    out = _sc_gather_scale()(K, idx, conf)
    return out.reshape(orig_shape)
